# baseline (speedup 1.0000x reference)
import jax
import jax.numpy as jnp
from jax.experimental import pallas as pl
from jax.experimental.pallas import tpu as pltpu

B = 512
L = 64
NB = 8
FEAT = 39
BOND = 10
FP = 64
RADIUS = 2
TT = 2
OUT_UNITS = 128
OUT_DIM = 1

BM = 8

_NEG = -9.0e8
_BN_SCALE = 1.0 / (1.0 + 1e-5) ** 0.5


def _leaky(x):
    return jnp.where(x >= 0, x, 0.01 * x)


def _elu(x):
    return jnp.where(x > 0, x, jnp.exp(jnp.minimum(x, 0.0)) - 1.0)


def _mm_t(x, w):
    return jax.lax.dot_general(
        x, w, (((1,), (1,)), ((), ())), preferred_element_type=jnp.float32
    )


def _mm(x, w):
    return jax.lax.dot_general(
        x, w, (((1,), (0,)), ((), ())), preferred_element_type=jnp.float32
    )


def _gru(x, h, wih, whh, bih, bhh):
    gi = _mm_t(x, wih) + bih
    gh = _mm_t(h, whh) + bhh
    r = jax.nn.sigmoid(gi[:, :FP] + gh[:, :FP])
    z = jax.nn.sigmoid(gi[:, FP:2 * FP] + gh[:, FP:2 * FP])
    n = jnp.tanh(gi[:, 2 * FP:] + r * gh[:, 2 * FP:])
    return (1.0 - z) * n + z * h


def _fused(x_atom_ref, x_bond_ref, idx_a_ref, idx_b_ref, mask_ref,
           atom_fc_W_ref, atom_fc_b_ref, neighbor_fc_W_ref, neighbor_fc_b_ref,
           gru_Wih_ref, gru_Whh_ref, gru_bih_ref, gru_bhh_ref,
           align_W_ref, align_b_ref, attend_W_ref, attend_b_ref,
           mol_gru_Wih_ref, mol_gru_Whh_ref, mol_gru_bih_ref, mol_gru_bhh_ref,
           mol_align_W_ref, mol_align_b_ref, mol_attend_W_ref, mol_attend_b_ref,
           bn_gamma_ref, bn_beta_ref, mol_output_W_ref, mol_output_b_ref,
           output_W_ref, output_b_ref, out_x_ref, out_y_ref):
    f32 = jnp.float32
    xa = jnp.reshape(x_atom_ref[...], (BM * L, FEAT))
    xb = jnp.reshape(x_bond_ref[...], (BM * L, BOND))
    ia = idx_a_ref[...]
    ib = idx_b_ref[...]
    mask = mask_ref[...]

    af = _leaky(_mm_t(xa, atom_fc_W_ref[...]) + atom_fc_b_ref[...])
    nb_W = neighbor_fc_W_ref[...]
    pa = _mm_t(xa, nb_W[:, :FEAT])
    pb = _mm_t(xb, nb_W[:, FEAT:])
    pa = jnp.reshape(pa, (BM, L, FP))
    pb = jnp.reshape(pb, (BM, L, FP))

    iota = jax.lax.broadcasted_iota(jnp.int32, (BM, L, NB, L), 3)
    oh_a = (ia[..., None] == iota).astype(f32)
    oh_b = (ib[..., None] == iota).astype(f32)
    oh = jnp.reshape(jnp.concatenate([oh_a, oh_b], axis=-1), (BM, L * NB, 2 * L))
    oh_a = jnp.reshape(oh_a, (BM, L * NB, L))

    nf_rows = []
    for m in range(BM):
        table = jnp.concatenate([pa[m], pb[m]], axis=0)
        nf_rows.append(_mm(oh[m], table))
    nf = _leaky(jnp.stack(nf_rows, axis=0) + neighbor_fc_b_ref[...])
    nf = jnp.reshape(nf, (BM, L, NB, FP))

    amask = (ia != L - 1).astype(f32)
    smask = jnp.where(ia == L - 1, _NEG, 0.0).astype(f32)

    align_W = align_W_ref[...]
    align_b = align_b_ref[...]
    attend_W = attend_W_ref[...]
    attend_b = attend_b_ref[...]

    w1 = align_W[0, 0, :FP]
    w2 = align_W[0, 0, FP:]
    s_self = jnp.sum(jnp.reshape(af, (BM, L, FP)) * w1[None, None, :], axis=-1)
    s_nb = jnp.sum(nf * w2[None, None, None, :], axis=-1)
    score = _leaky(s_self[:, :, None] + s_nb + align_b[0, 0]) + smask
    mx = jnp.max(score, axis=-1, keepdims=True)
    ex = jnp.exp(score - mx)
    w = ex / jnp.sum(ex, axis=-1, keepdims=True) * amask
    ctx_pre = jnp.reshape(jnp.sum(w[..., None] * nf, axis=2), (BM * L, FP))
    wsum = jnp.reshape(jnp.sum(w, axis=-1), (BM * L, 1))
    ctx = _elu(_mm_t(ctx_pre, attend_W[0]) + wsum * attend_b[0][None, :])
    h = _gru(ctx, af, gru_Wih_ref[...][0], gru_Whh_ref[...][0],
             gru_bih_ref[...][0], gru_bhh_ref[...][0])
    act = jax.nn.relu(h)

    actr = jnp.reshape(act, (BM, L, FP))
    w1 = align_W[1, 0, :FP]
    w2 = align_W[1, 0, FP:]
    s_self = jnp.sum(actr * w1[None, None, :], axis=-1)
    t = jnp.sum(actr * w2[None, None, :], axis=-1)
    s_nb = jnp.sum(jnp.reshape(oh_a, (BM, L, NB, L)) * t[:, None, None, :], axis=-1)
    score = _leaky(s_self[:, :, None] + s_nb + align_b[1, 0]) + smask
    mx = jnp.max(score, axis=-1, keepdims=True)
    ex = jnp.exp(score - mx)
    w = ex / jnp.sum(ex, axis=-1, keepdims=True) * amask
    scat = jnp.sum(jnp.reshape(oh_a, (BM, L, NB, L)) * w[..., None], axis=2)
    ctx_rows = []
    for m in range(BM):
        ctx_rows.append(_mm(scat[m], actr[m]))
    ctx_pre = jnp.reshape(jnp.stack(ctx_rows, axis=0), (BM * L, FP))
    wsum = jnp.reshape(jnp.sum(w, axis=-1), (BM * L, 1))
    ctx = _elu(_mm_t(ctx_pre, attend_W[1]) + wsum * attend_b[1][None, :])
    h = _gru(ctx, h, gru_Wih_ref[...][1], gru_Whh_ref[...][1],
             gru_bih_ref[...][1], gru_bhh_ref[...][1])
    act = jax.nn.relu(h)

    actr = jnp.reshape(act, (BM, L, FP))
    mol_f = jnp.sum(actr * mask[..., None], axis=1)
    act_mol = jax.nn.relu(mol_f)
    mol_smask = jnp.where(mask == 0, _NEG, 0.0).astype(f32)

    mw = mol_align_W_ref[...]
    mw1 = mw[0, :FP]
    mw2 = mw[0, FP:]
    mb = mol_align_b_ref[...]
    m_attend_W = mol_attend_W_ref[...]
    m_attend_b = mol_attend_b_ref[...]
    s_act = jnp.sum(actr * mw2[None, None, :], axis=-1)
    gamma = bn_gamma_ref[...]
    beta = bn_beta_ref[...]
    for _ in range(TT):
        s_mol = jnp.sum(act_mol * mw1[None, :], axis=-1, keepdims=True)
        score = _leaky(s_mol + s_act + mb[0, 0]) + mol_smask
        mx = jnp.max(score, axis=-1, keepdims=True)
        ex = jnp.exp(score - mx)
        w = ex / jnp.sum(ex, axis=-1, keepdims=True) * mask
        ctx_pre = jnp.sum(w[..., None] * actr, axis=1)
        wsum = jnp.sum(w, axis=-1, keepdims=True)
        mol_ctx = _elu(_mm_t(ctx_pre, m_attend_W) + wsum * m_attend_b)
        mol_ctx = mol_ctx * _BN_SCALE * gamma + beta
        mol_f = _gru(mol_ctx, mol_f, mol_gru_Wih_ref[...], mol_gru_Whh_ref[...],
                     mol_gru_bih_ref[...], mol_gru_bhh_ref[...])
        act_mol = jax.nn.relu(mol_f)

    mol_pred = _mm_t(mol_f, mol_output_W_ref[...]) + mol_output_b_ref[...]
    xo = jnp.sum(mol_pred * output_W_ref[...][0][None, :], axis=-1,
                 keepdims=True) + output_b_ref[...][0, 0]
    out_x_ref[...] = xo
    out_y_ref[...] = jax.nn.sigmoid(xo)


def _full(shape):
    nd = len(shape)
    return pl.BlockSpec(shape, lambda i: (0,) * nd)


def kernel(x_atom, x_bond, x_atom_index, x_bond_index, x_mask, x_chemical_info,
           atom_fc_W, atom_fc_b, neighbor_fc_W, neighbor_fc_b,
           gru_Wih, gru_Whh, gru_bih, gru_bhh, align_W, align_b,
           attend_W, attend_b, mol_gru_Wih, mol_gru_Whh, mol_gru_bih,
           mol_gru_bhh, mol_align_W, mol_align_b, mol_attend_W, mol_attend_b,
           bn_gamma, bn_beta, mol_output_W, mol_output_b, output_W, output_b):
    del x_chemical_info
    ia = x_atom_index.astype(jnp.int32)
    ib = x_bond_index.astype(jnp.int32)
    r2 = lambda v: jnp.reshape(v, (1, -1))

    grid = (B // BM,)
    in_specs = [
        pl.BlockSpec((BM, L, FEAT), lambda i: (i, 0, 0)),
        pl.BlockSpec((BM, L, BOND), lambda i: (i, 0, 0)),
        pl.BlockSpec((BM, L, NB), lambda i: (i, 0, 0)),
        pl.BlockSpec((BM, L, NB), lambda i: (i, 0, 0)),
        pl.BlockSpec((BM, L), lambda i: (i, 0)),
        _full((FP, FEAT)), _full((1, FP)),
        _full((FP, FEAT + BOND)), _full((1, FP)),
        _full((RADIUS, 3 * FP, FP)), _full((RADIUS, 3 * FP, FP)),
        _full((RADIUS, 3 * FP)), _full((RADIUS, 3 * FP)),
        _full((RADIUS, 1, 2 * FP)), _full((RADIUS, 1)),
        _full((RADIUS, FP, FP)), _full((RADIUS, FP)),
        _full((3 * FP, FP)), _full((3 * FP, FP)),
        _full((1, 3 * FP)), _full((1, 3 * FP)),
        _full((1, 2 * FP)), _full((1, 1)),
        _full((FP, FP)), _full((1, FP)),
        _full((1, FP)), _full((1, FP)),
        _full((OUT_UNITS, FP)), _full((1, OUT_UNITS)),
        _full((OUT_DIM, OUT_UNITS)), _full((1, OUT_DIM)),
    ]
    out_specs = [
        pl.BlockSpec((BM, OUT_DIM), lambda i: (i, 0)),
        pl.BlockSpec((BM, OUT_DIM), lambda i: (i, 0)),
    ]
    out_shape = [
        jax.ShapeDtypeStruct((B, OUT_DIM), jnp.float32),
        jax.ShapeDtypeStruct((B, OUT_DIM), jnp.float32),
    ]
    xo, yo = pl.pallas_call(
        _fused,
        grid=grid,
        in_specs=in_specs,
        out_specs=out_specs,
        out_shape=out_shape,
        compiler_params=pltpu.CompilerParams(
            dimension_semantics=("parallel",),
        ),
    )(x_atom, x_bond, ia, ib, x_mask,
      atom_fc_W, r2(atom_fc_b), neighbor_fc_W, r2(neighbor_fc_b),
      gru_Wih, gru_Whh, gru_bih, gru_bhh, align_W, align_b,
      attend_W, attend_b, mol_gru_Wih, mol_gru_Whh,
      r2(mol_gru_bih), r2(mol_gru_bhh), mol_align_W, r2(mol_align_b),
      mol_attend_W, r2(mol_attend_b), r2(bn_gamma), r2(bn_beta),
      mol_output_W, r2(mol_output_b), output_W, r2(output_b))
    return (xo, yo)


# speedup vs baseline: 1.2828x; 1.2828x over previous
import jax
import jax.numpy as jnp
from jax.experimental import pallas as pl
from jax.experimental.pallas import tpu as pltpu

B = 512
L = 64
NB = 8
FEAT = 39
BOND = 10
FP = 64
RADIUS = 2
TT = 2
OUT_UNITS = 128
OUT_DIM = 1

BM = 16

_NEG = -9.0e8
_BN_SCALE = 1.0 / (1.0 + 1e-5) ** 0.5


def _leaky(x):
    return jnp.where(x >= 0, x, 0.01 * x)


def _elu(x):
    return jnp.where(x > 0, x, jnp.exp(jnp.minimum(x, 0.0)) - 1.0)


def _mm_t(x, w):
    return jax.lax.dot_general(
        x, w, (((1,), (1,)), ((), ())), preferred_element_type=jnp.float32
    )


def _mm(x, w):
    return jax.lax.dot_general(
        x, w, (((1,), (0,)), ((), ())), preferred_element_type=jnp.float32
    )


def _gru(x, h, wih, whh, bih, bhh):
    gi = _mm_t(x, wih) + bih
    gh = _mm_t(h, whh) + bhh
    r = jax.nn.sigmoid(gi[:, :FP] + gh[:, :FP])
    z = jax.nn.sigmoid(gi[:, FP:2 * FP] + gh[:, FP:2 * FP])
    n = jnp.tanh(gi[:, 2 * FP:] + r * gh[:, 2 * FP:])
    return (1.0 - z) * n + z * h


def _fused(x_atom_ref, x_bond_ref, idx_a_ref, idx_b_ref, mask_ref,
           atom_fc_W_ref, atom_fc_b_ref, neighbor_fc_W_ref, neighbor_fc_b_ref,
           gru_Wih_ref, gru_Whh_ref, gru_bih_ref, gru_bhh_ref,
           align_W_ref, align_b_ref, attend_W_ref, attend_b_ref,
           mol_gru_Wih_ref, mol_gru_Whh_ref, mol_gru_bih_ref, mol_gru_bhh_ref,
           mol_align_W_ref, mol_align_b_ref, mol_attend_W_ref, mol_attend_b_ref,
           bn_gamma_ref, bn_beta_ref, mol_output_W_ref, mol_output_b_ref,
           output_W_ref, output_b_ref, out_x_ref, out_y_ref):
    f32 = jnp.float32
    xa = jnp.reshape(x_atom_ref[...], (BM * L, FEAT))
    xb = jnp.reshape(x_bond_ref[...], (BM * L, BOND))
    ia = idx_a_ref[...]
    ib = idx_b_ref[...]
    mask = mask_ref[...]

    af = _leaky(_mm_t(xa, atom_fc_W_ref[...]) + atom_fc_b_ref[...])
    nb_W = neighbor_fc_W_ref[...]
    pa = _mm_t(xa, nb_W[:, :FEAT])
    pb = _mm_t(xb, nb_W[:, FEAT:])
    pa = jnp.reshape(pa, (BM, L, FP))
    pb = jnp.reshape(pb, (BM, L, FP))

    iota = jax.lax.broadcasted_iota(jnp.int32, (BM, L, NB, L), 3)
    oh_a = (ia[..., None] == iota).astype(f32)
    oh_b = (ib[..., None] == iota).astype(f32)
    oh = jnp.reshape(jnp.concatenate([oh_a, oh_b], axis=-1), (BM, L * NB, 2 * L))
    oh_a = jnp.reshape(oh_a, (BM, L * NB, L))

    nf_rows = []
    for m in range(BM):
        table = jnp.concatenate([pa[m], pb[m]], axis=0)
        nf_rows.append(_mm(oh[m], table))
    nf = _leaky(jnp.stack(nf_rows, axis=0) + neighbor_fc_b_ref[...])
    nf = jnp.reshape(nf, (BM, L, NB, FP))

    amask = (ia != L - 1).astype(f32)
    smask = jnp.where(ia == L - 1, _NEG, 0.0).astype(f32)

    align_W = align_W_ref[...]
    align_b = align_b_ref[...]
    attend_W = attend_W_ref[...]
    attend_b = attend_b_ref[...]

    w1 = align_W[0, 0, :FP]
    w2 = align_W[0, 0, FP:]
    s_self = jnp.sum(jnp.reshape(af, (BM, L, FP)) * w1[None, None, :], axis=-1)
    s_nb = jnp.sum(nf * w2[None, None, None, :], axis=-1)
    score = _leaky(s_self[:, :, None] + s_nb + align_b[0, 0]) + smask
    mx = jnp.max(score, axis=-1, keepdims=True)
    ex = jnp.exp(score - mx)
    w = ex / jnp.sum(ex, axis=-1, keepdims=True) * amask
    ctx_pre = jnp.reshape(jnp.sum(w[..., None] * nf, axis=2), (BM * L, FP))
    wsum = jnp.reshape(jnp.sum(w, axis=-1), (BM * L, 1))
    ctx = _elu(_mm_t(ctx_pre, attend_W[0]) + wsum * attend_b[0][None, :])
    h = _gru(ctx, af, gru_Wih_ref[...][0], gru_Whh_ref[...][0],
             gru_bih_ref[...][0], gru_bhh_ref[...][0])
    act = jax.nn.relu(h)

    actr = jnp.reshape(act, (BM, L, FP))
    w1 = align_W[1, 0, :FP]
    w2 = align_W[1, 0, FP:]
    s_self = jnp.sum(actr * w1[None, None, :], axis=-1)
    t = jnp.sum(actr * w2[None, None, :], axis=-1)
    s_nb = jnp.sum(jnp.reshape(oh_a, (BM, L, NB, L)) * t[:, None, None, :], axis=-1)
    score = _leaky(s_self[:, :, None] + s_nb + align_b[1, 0]) + smask
    mx = jnp.max(score, axis=-1, keepdims=True)
    ex = jnp.exp(score - mx)
    w = ex / jnp.sum(ex, axis=-1, keepdims=True) * amask
    scat = jnp.sum(jnp.reshape(oh_a, (BM, L, NB, L)) * w[..., None], axis=2)
    ctx_rows = []
    for m in range(BM):
        ctx_rows.append(_mm(scat[m], actr[m]))
    ctx_pre = jnp.reshape(jnp.stack(ctx_rows, axis=0), (BM * L, FP))
    wsum = jnp.reshape(jnp.sum(w, axis=-1), (BM * L, 1))
    ctx = _elu(_mm_t(ctx_pre, attend_W[1]) + wsum * attend_b[1][None, :])
    h = _gru(ctx, h, gru_Wih_ref[...][1], gru_Whh_ref[...][1],
             gru_bih_ref[...][1], gru_bhh_ref[...][1])
    act = jax.nn.relu(h)

    actr = jnp.reshape(act, (BM, L, FP))
    mol_f = jnp.sum(actr * mask[..., None], axis=1)
    act_mol = jax.nn.relu(mol_f)
    mol_smask = jnp.where(mask == 0, _NEG, 0.0).astype(f32)

    mw = mol_align_W_ref[...]
    mw1 = mw[0, :FP]
    mw2 = mw[0, FP:]
    mb = mol_align_b_ref[...]
    m_attend_W = mol_attend_W_ref[...]
    m_attend_b = mol_attend_b_ref[...]
    s_act = jnp.sum(actr * mw2[None, None, :], axis=-1)
    gamma = bn_gamma_ref[...]
    beta = bn_beta_ref[...]
    for _ in range(TT):
        s_mol = jnp.sum(act_mol * mw1[None, :], axis=-1, keepdims=True)
        score = _leaky(s_mol + s_act + mb[0, 0]) + mol_smask
        mx = jnp.max(score, axis=-1, keepdims=True)
        ex = jnp.exp(score - mx)
        w = ex / jnp.sum(ex, axis=-1, keepdims=True) * mask
        ctx_pre = jnp.sum(w[..., None] * actr, axis=1)
        wsum = jnp.sum(w, axis=-1, keepdims=True)
        mol_ctx = _elu(_mm_t(ctx_pre, m_attend_W) + wsum * m_attend_b)
        mol_ctx = mol_ctx * _BN_SCALE * gamma + beta
        mol_f = _gru(mol_ctx, mol_f, mol_gru_Wih_ref[...], mol_gru_Whh_ref[...],
                     mol_gru_bih_ref[...], mol_gru_bhh_ref[...])
        act_mol = jax.nn.relu(mol_f)

    mol_pred = _mm_t(mol_f, mol_output_W_ref[...]) + mol_output_b_ref[...]
    xo = jnp.sum(mol_pred * output_W_ref[...][0][None, :], axis=-1,
                 keepdims=True) + output_b_ref[...][0, 0]
    out_x_ref[...] = xo
    out_y_ref[...] = jax.nn.sigmoid(xo)


def _full(shape):
    nd = len(shape)
    return pl.BlockSpec(shape, lambda i: (0,) * nd)


def kernel(x_atom, x_bond, x_atom_index, x_bond_index, x_mask, x_chemical_info,
           atom_fc_W, atom_fc_b, neighbor_fc_W, neighbor_fc_b,
           gru_Wih, gru_Whh, gru_bih, gru_bhh, align_W, align_b,
           attend_W, attend_b, mol_gru_Wih, mol_gru_Whh, mol_gru_bih,
           mol_gru_bhh, mol_align_W, mol_align_b, mol_attend_W, mol_attend_b,
           bn_gamma, bn_beta, mol_output_W, mol_output_b, output_W, output_b):
    del x_chemical_info
    ia = x_atom_index.astype(jnp.int32)
    ib = x_bond_index.astype(jnp.int32)
    r2 = lambda v: jnp.reshape(v, (1, -1))

    grid = (B // BM,)
    in_specs = [
        pl.BlockSpec((BM, L, FEAT), lambda i: (i, 0, 0)),
        pl.BlockSpec((BM, L, BOND), lambda i: (i, 0, 0)),
        pl.BlockSpec((BM, L, NB), lambda i: (i, 0, 0)),
        pl.BlockSpec((BM, L, NB), lambda i: (i, 0, 0)),
        pl.BlockSpec((BM, L), lambda i: (i, 0)),
        _full((FP, FEAT)), _full((1, FP)),
        _full((FP, FEAT + BOND)), _full((1, FP)),
        _full((RADIUS, 3 * FP, FP)), _full((RADIUS, 3 * FP, FP)),
        _full((RADIUS, 3 * FP)), _full((RADIUS, 3 * FP)),
        _full((RADIUS, 1, 2 * FP)), _full((RADIUS, 1)),
        _full((RADIUS, FP, FP)), _full((RADIUS, FP)),
        _full((3 * FP, FP)), _full((3 * FP, FP)),
        _full((1, 3 * FP)), _full((1, 3 * FP)),
        _full((1, 2 * FP)), _full((1, 1)),
        _full((FP, FP)), _full((1, FP)),
        _full((1, FP)), _full((1, FP)),
        _full((OUT_UNITS, FP)), _full((1, OUT_UNITS)),
        _full((OUT_DIM, OUT_UNITS)), _full((1, OUT_DIM)),
    ]
    out_specs = [
        pl.BlockSpec((BM, OUT_DIM), lambda i: (i, 0)),
        pl.BlockSpec((BM, OUT_DIM), lambda i: (i, 0)),
    ]
    out_shape = [
        jax.ShapeDtypeStruct((B, OUT_DIM), jnp.float32),
        jax.ShapeDtypeStruct((B, OUT_DIM), jnp.float32),
    ]
    xo, yo = pl.pallas_call(
        _fused,
        grid=grid,
        in_specs=in_specs,
        out_specs=out_specs,
        out_shape=out_shape,
        compiler_params=pltpu.CompilerParams(
            dimension_semantics=("parallel",),
        ),
    )(x_atom, x_bond, ia, ib, x_mask,
      atom_fc_W, r2(atom_fc_b), neighbor_fc_W, r2(neighbor_fc_b),
      gru_Wih, gru_Whh, gru_bih, gru_bhh, align_W, align_b,
      attend_W, attend_b, mol_gru_Wih, mol_gru_Whh,
      r2(mol_gru_bih), r2(mol_gru_bhh), mol_align_W, r2(mol_align_b),
      mol_attend_W, r2(mol_attend_b), r2(bn_gamma), r2(bn_beta),
      mol_output_W, r2(mol_output_b), output_W, r2(output_b))
    return (xo, yo)
